# Initial kernel scaffold; baseline (speedup 1.0000x reference)
import jax
import jax.numpy as jnp
from jax.experimental import pallas as pl


def kernel(x, edge_index, embedding, W, att_i, att_j, att_em_i, att_em_j, bias, gamma, beta):
    raise NotImplementedError("write your pallas kernel here")


# speedup vs baseline: 6.3308x; 6.3308x over previous
import dataclasses
import functools

import jax
import jax.numpy as jnp
from jax import lax
from jax.experimental import pallas as pl
from jax.experimental.pallas import tpu as pltpu
from jax.experimental.pallas import tpu_sc as plsc

N = 10000
E = 320000
D = 128
DW = 128
NCORES = 2
NSUB = 16
NTILES = NCORES * NSUB
CHUNK = 64
NCHUNKS = E // CHUNK
ROWS_PER_TILE = 624
TAIL_ROWS = N - NSUB * ROWS_PER_TILE
NEG_SLOPE = 0.2
ROW_BLOCK = 1000


def _prologue_body(x_ref, emb_ref, wt_ref, vix_ref, vie_ref, vjx_ref, vje_ref,
                   xlin_ref, ai_ref, aj_ref):
    xl = jnp.dot(x_ref[...], wt_ref[...], preferred_element_type=jnp.float32)
    xlin_ref[...] = xl
    emb = emb_ref[...]
    ai_ref[...] = (jnp.sum(xl * vix_ref[...], axis=1, keepdims=True)
                   + jnp.sum(emb * vie_ref[...], axis=1, keepdims=True))
    aj_ref[...] = (jnp.sum(xl * vjx_ref[...], axis=1, keepdims=True)
                   + jnp.sum(emb * vje_ref[...], axis=1, keepdims=True))


def _prologue(x, emb, wt, vix, vie, vjx, vje):
    grid = (N // ROW_BLOCK,)
    return pl.pallas_call(
        _prologue_body,
        grid=grid,
        in_specs=[
            pl.BlockSpec((ROW_BLOCK, D), lambda j: (j, 0)),
            pl.BlockSpec((ROW_BLOCK, D), lambda j: (j, 0)),
            pl.BlockSpec((D, D), lambda j: (0, 0)),
            pl.BlockSpec((1, D), lambda j: (0, 0)),
            pl.BlockSpec((1, D), lambda j: (0, 0)),
            pl.BlockSpec((1, D), lambda j: (0, 0)),
            pl.BlockSpec((1, D), lambda j: (0, 0)),
        ],
        out_specs=[
            pl.BlockSpec((ROW_BLOCK, D), lambda j: (j, 0)),
            pl.BlockSpec((ROW_BLOCK, 1), lambda j: (j, 0)),
            pl.BlockSpec((ROW_BLOCK, 1), lambda j: (j, 0)),
        ],
        out_shape=[
            jax.ShapeDtypeStruct((N, D), jnp.float32),
            jax.ShapeDtypeStruct((N, 1), jnp.float32),
            jax.ShapeDtypeStruct((N, 1), jnp.float32),
        ],
    )(x, emb, wt, vix, vie, vjx, vje)


def _sc_edge(xlin, src, dst, ai, aj):
    mesh = plsc.VectorSubcoreMesh(core_axis_name="c", subcore_axis_name="s")
    cp = pltpu.CompilerParams()
    if "needs_layout_passes" in pltpu.CompilerParams.__dataclass_fields__:
        cp = dataclasses.replace(cp, needs_layout_passes=False)

    @functools.partial(
        pl.kernel,
        compiler_params=cp,
        out_type=jax.ShapeDtypeStruct((NCORES, N, DW), jnp.float32),
        mesh=mesh,
        scratch_types=[
            pltpu.VMEM_SHARED((N, DW), jnp.float32),
            pltpu.VMEM((N,), jnp.float32),
            pltpu.VMEM((N,), jnp.float32),
            pltpu.VMEM((CHUNK,), jnp.int32),
            pltpu.VMEM((CHUNK,), jnp.int32),
            pltpu.VMEM((CHUNK, D), jnp.float32),
            pltpu.VMEM((CHUNK, DW), jnp.float32),
            pltpu.VMEM((CHUNK,), jnp.float32),
            pltpu.SemaphoreType.DMA,
        ],
    )
    def sc_kernel(xlin_hbm, src_hbm, dst_hbm, ai_hbm, aj_hbm, nd_hbm,
                  acc_sh, ai_v, aj_v, src_v, dst_v, rows_v, out_v, w_v, sem):
        cid = lax.axis_index("c")
        sid = lax.axis_index("s")
        wid = sid * NCORES + cid

        zeros16 = jnp.zeros((16,), jnp.float32)

        @pl.loop(0, CHUNK)
        def _(r):
            for k in range(DW // 16):
                out_v[r, pl.ds(k * 16, 16)] = zeros16

        base_r = sid * ROWS_PER_TILE
        nfull = ROWS_PER_TILE // CHUNK
        tail = ROWS_PER_TILE - nfull * CHUNK
        for q in range(nfull):
            pltpu.sync_copy(out_v, acc_sh.at[pl.ds(base_r + q * CHUNK, CHUNK)])
        pltpu.sync_copy(out_v.at[pl.ds(0, tail)],
                        acc_sh.at[pl.ds(base_r + nfull * CHUNK, tail)])

        @pl.when(sid == NSUB - 1)
        def _():
            pltpu.sync_copy(out_v.at[pl.ds(0, TAIL_ROWS)],
                            acc_sh.at[pl.ds(NSUB * ROWS_PER_TILE, TAIL_ROWS)])

        pltpu.sync_copy(ai_hbm, ai_v)
        pltpu.sync_copy(aj_hbm, aj_v)

        plsc.subcore_barrier()

        nchunks = jnp.where(wid < NCHUNKS - (NCHUNKS // NTILES) * NTILES,
                            NCHUNKS // NTILES + 1, NCHUNKS // NTILES)

        def chunk_body(k, _):
            chunk = k * NTILES + wid
            base = chunk * CHUNK
            pltpu.sync_copy(src_hbm.at[pl.ds(base, CHUNK)], src_v)
            pltpu.sync_copy(dst_hbm.at[pl.ds(base, CHUNK)], dst_v)
            gcopy = pltpu.async_copy(xlin_hbm.at[src_v], rows_v, sem)
            for b in range(CHUNK // 16):
                sl = pl.ds(b * 16, 16)
                si = src_v[sl]
                di = dst_v[sl]
                s = (plsc.load_gather(ai_v, [di])
                     + plsc.load_gather(aj_v, [si]))
                s = jnp.maximum(s, s * NEG_SLOPE)
                w = jnp.where(si == di, 0.0, jnp.exp(s))
                w_v[sl] = w
            gcopy.wait()

            @pl.loop(0, CHUNK)
            def _(e):
                wb = plsc.load_gather(w_v, [jnp.full((16,), e, jnp.int32)])
                for k2 in range(D // 16):
                    sle = pl.ds(k2 * 16, 16)
                    out_v[e, sle] = rows_v[e, sle] * wb

            pltpu.sync_copy(out_v, acc_sh.at[dst_v], add=True)
            return ()

        lax.fori_loop(0, nchunks, chunk_body, ())

        plsc.subcore_barrier()

        def flush(off, nrows):
            pltpu.sync_copy(acc_sh.at[pl.ds(off, nrows)],
                            out_v.at[pl.ds(0, nrows)])
            pltpu.sync_copy(out_v.at[pl.ds(0, nrows)],
                            nd_hbm.at[cid, pl.ds(off, nrows)])

        for q in range(nfull):
            flush(base_r + q * CHUNK, CHUNK)
        flush(base_r + nfull * CHUNK, tail)

        @pl.when(sid == NSUB - 1)
        def _():
            flush(NSUB * ROWS_PER_TILE, TAIL_ROWS)

    return sc_kernel(xlin, src, dst, ai, aj)


def _sc_denom(src, dst, ai, aj):
    mesh = plsc.VectorSubcoreMesh(core_axis_name="c", subcore_axis_name="s")
    cp = pltpu.CompilerParams()
    if "needs_layout_passes" in pltpu.CompilerParams.__dataclass_fields__:
        cp = dataclasses.replace(cp, needs_layout_passes=False)

    @functools.partial(
        pl.kernel,
        compiler_params=cp,
        out_type=jax.ShapeDtypeStruct((NCORES, N, 16), jnp.float32),
        mesh=mesh,
        scratch_types=[
            pltpu.VMEM_SHARED((N, 16), jnp.float32),
            pltpu.VMEM((N,), jnp.float32),
            pltpu.VMEM((N,), jnp.float32),
            pltpu.VMEM((CHUNK,), jnp.int32),
            pltpu.VMEM((CHUNK,), jnp.int32),
            pltpu.VMEM((CHUNK, 16), jnp.float32),
            pltpu.VMEM((CHUNK,), jnp.float32),
        ],
    )
    def den_kernel(src_hbm, dst_hbm, ai_hbm, aj_hbm, den_hbm,
                   den_sh, ai_v, aj_v, src_v, dst_v, w16_v, w_v):
        cid = lax.axis_index("c")
        sid = lax.axis_index("s")
        wid = sid * NCORES + cid

        zeros16 = jnp.zeros((16,), jnp.float32)

        @pl.loop(0, CHUNK)
        def _(r):
            w16_v[r, :] = zeros16

        base_r = sid * ROWS_PER_TILE
        nfull = ROWS_PER_TILE // CHUNK
        tail = ROWS_PER_TILE - nfull * CHUNK
        for q in range(nfull):
            pltpu.sync_copy(w16_v, den_sh.at[pl.ds(base_r + q * CHUNK, CHUNK)])
        pltpu.sync_copy(w16_v.at[pl.ds(0, tail)],
                        den_sh.at[pl.ds(base_r + nfull * CHUNK, tail)])

        @pl.when(sid == NSUB - 1)
        def _():
            pltpu.sync_copy(w16_v.at[pl.ds(0, TAIL_ROWS)],
                            den_sh.at[pl.ds(NSUB * ROWS_PER_TILE, TAIL_ROWS)])

        pltpu.sync_copy(ai_hbm, ai_v)
        pltpu.sync_copy(aj_hbm, aj_v)

        plsc.subcore_barrier()

        nchunks = jnp.where(wid < NCHUNKS - (NCHUNKS // NTILES) * NTILES,
                            NCHUNKS // NTILES + 1, NCHUNKS // NTILES)

        def chunk_body(k, _):
            chunk = k * NTILES + wid
            base = chunk * CHUNK
            pltpu.sync_copy(src_hbm.at[pl.ds(base, CHUNK)], src_v)
            pltpu.sync_copy(dst_hbm.at[pl.ds(base, CHUNK)], dst_v)
            for b in range(CHUNK // 16):
                sl = pl.ds(b * 16, 16)
                si = src_v[sl]
                di = dst_v[sl]
                s = (plsc.load_gather(ai_v, [di])
                     + plsc.load_gather(aj_v, [si]))
                s = jnp.maximum(s, s * NEG_SLOPE)
                w = jnp.where(si == di, 0.0, jnp.exp(s))
                w_v[sl] = w

            @pl.loop(0, CHUNK)
            def _(e):
                w16_v[e, :] = plsc.load_gather(
                    w_v, [jnp.full((16,), e, jnp.int32)])

            pltpu.sync_copy(w16_v, den_sh.at[dst_v], add=True)
            return ()

        lax.fori_loop(0, nchunks, chunk_body, ())

        plsc.subcore_barrier()

        def flush(off, nrows):
            pltpu.sync_copy(den_sh.at[pl.ds(off, nrows)],
                            w16_v.at[pl.ds(0, nrows)])
            pltpu.sync_copy(w16_v.at[pl.ds(0, nrows)],
                            den_hbm.at[cid, pl.ds(off, nrows)])

        for q in range(nfull):
            flush(base_r + q * CHUNK, CHUNK)
        flush(base_r + nfull * CHUNK, tail)

        @pl.when(sid == NSUB - 1)
        def _():
            flush(NSUB * ROWS_PER_TILE, TAIL_ROWS)

    return den_kernel(src, dst, ai, aj)


def _epilogue_body(xlin_ref, ai_ref, aj_ref, nd0_ref, nd1_ref,
                   d0_ref, d1_ref, bias_ref, gamma_ref, beta_ref, out_ref):
    s = ai_ref[...] + aj_ref[...]
    s = jnp.maximum(s, s * NEG_SLOPE)
    wself = jnp.exp(s)
    nd0 = nd0_ref[...]
    nd1 = nd1_ref[...]
    numer = nd0[:, :D] + nd1[:, :D] + wself * xlin_ref[...]
    den = d0_ref[...][:, :1] + d1_ref[...][:, :1] + wself
    out = numer / den
    inv_bn = 1.0 / jnp.sqrt(1.0 + 1e-5)
    out = (out + bias_ref[...]) * inv_bn * gamma_ref[...] + beta_ref[...]
    out_ref[...] = jnp.maximum(out, 0.0)


def _epilogue(xlin, ai, aj, nd0, nd1, d0, d1, bias, gamma, beta):
    grid = (N // ROW_BLOCK,)
    row_spec = pl.BlockSpec((ROW_BLOCK, D), lambda j: (j, 0))
    nd_spec = pl.BlockSpec((ROW_BLOCK, DW), lambda j: (j, 0))
    d_spec = pl.BlockSpec((ROW_BLOCK, DW), lambda j: (j, 0))
    col_spec = pl.BlockSpec((ROW_BLOCK, 1), lambda j: (j, 0))
    vec_spec = pl.BlockSpec((1, D), lambda j: (0, 0))
    return pl.pallas_call(
        _epilogue_body,
        grid=grid,
        in_specs=[row_spec, col_spec, col_spec, nd_spec, nd_spec,
                  d_spec, d_spec, vec_spec, vec_spec, vec_spec],
        out_specs=row_spec,
        out_shape=jax.ShapeDtypeStruct((N, D), jnp.float32),
    )(xlin, ai, aj, nd0, nd1, d0, d1, bias, gamma, beta)


def kernel(x, edge_index, embedding, W, att_i, att_j, att_em_i, att_em_j,
           bias, gamma, beta):
    wt = W.T
    vix = att_i.reshape(1, D)
    vie = att_em_i.reshape(1, D)
    vjx = att_j.reshape(1, D)
    vje = att_em_j.reshape(1, D)

    xlin, ai, aj = _prologue(x, embedding, wt, vix, vie, vjx, vje)

    src = edge_index[0]
    dst = edge_index[1]
    nd = _sc_edge(xlin, src, dst, ai.reshape(N), aj.reshape(N))
    ones = jnp.ones((N, D), jnp.float32)
    den = _sc_edge(ones, src, dst, ai.reshape(N), aj.reshape(N))

    out = _epilogue(xlin, ai, aj, nd[0], nd[1], den[0], den[1],
                    bias.reshape(1, D), gamma.reshape(1, D),
                    beta.reshape(1, D))
    return out


# speedup vs baseline: 8.8482x; 1.3976x over previous
import dataclasses
import functools

import jax
import jax.numpy as jnp
from jax import lax
from jax.experimental import pallas as pl
from jax.experimental.pallas import tpu as pltpu
from jax.experimental.pallas import tpu_sc as plsc

N = 10000
E = 320000
D = 128
DW = 128
NCORES = 2
NSUB = 16
NTILES = NCORES * NSUB
CHUNK = 64
NCHUNKS = E // CHUNK
ROWS_PER_TILE = 624
TAIL_ROWS = N - NSUB * ROWS_PER_TILE
NEG_SLOPE = 0.2
ROW_BLOCK = 1000


def _prologue_body(x_ref, emb_ref, wt_ref, vix_ref, vie_ref, vjx_ref, vje_ref,
                   xlin_ref, ai_ref, aj_ref):
    xl = jnp.dot(x_ref[...], wt_ref[...], preferred_element_type=jnp.float32)
    xlin_ref[...] = xl
    emb = emb_ref[...]
    ai_ref[...] = (jnp.sum(xl * vix_ref[...], axis=1, keepdims=True)
                   + jnp.sum(emb * vie_ref[...], axis=1, keepdims=True))
    aj_ref[...] = (jnp.sum(xl * vjx_ref[...], axis=1, keepdims=True)
                   + jnp.sum(emb * vje_ref[...], axis=1, keepdims=True))


def _prologue(x, emb, wt, vix, vie, vjx, vje):
    grid = (N // ROW_BLOCK,)
    return pl.pallas_call(
        _prologue_body,
        grid=grid,
        in_specs=[
            pl.BlockSpec((ROW_BLOCK, D), lambda j: (j, 0)),
            pl.BlockSpec((ROW_BLOCK, D), lambda j: (j, 0)),
            pl.BlockSpec((D, D), lambda j: (0, 0)),
            pl.BlockSpec((1, D), lambda j: (0, 0)),
            pl.BlockSpec((1, D), lambda j: (0, 0)),
            pl.BlockSpec((1, D), lambda j: (0, 0)),
            pl.BlockSpec((1, D), lambda j: (0, 0)),
        ],
        out_specs=[
            pl.BlockSpec((ROW_BLOCK, D), lambda j: (j, 0)),
            pl.BlockSpec((ROW_BLOCK, 1), lambda j: (j, 0)),
            pl.BlockSpec((ROW_BLOCK, 1), lambda j: (j, 0)),
        ],
        out_shape=[
            jax.ShapeDtypeStruct((N, D), jnp.float32),
            jax.ShapeDtypeStruct((N, 1), jnp.float32),
            jax.ShapeDtypeStruct((N, 1), jnp.float32),
        ],
    )(x, emb, wt, vix, vie, vjx, vje)


def _sc_edge(xlin, src, dst, ai, aj):
    mesh = plsc.VectorSubcoreMesh(core_axis_name="c", subcore_axis_name="s")
    cp = pltpu.CompilerParams()
    if "needs_layout_passes" in pltpu.CompilerParams.__dataclass_fields__:
        cp = dataclasses.replace(cp, needs_layout_passes=False)

    @functools.partial(
        pl.kernel,
        compiler_params=cp,
        out_type=jax.ShapeDtypeStruct((NCORES, N, DW), jnp.float32),
        mesh=mesh,
        scratch_types=[
            pltpu.VMEM_SHARED((N, DW), jnp.float32),
            pltpu.VMEM((N,), jnp.float32),
            pltpu.VMEM((N,), jnp.float32),
            pltpu.VMEM((CHUNK,), jnp.int32),
            pltpu.VMEM((CHUNK,), jnp.int32),
            pltpu.VMEM((CHUNK, D), jnp.float32),
            pltpu.VMEM((CHUNK, DW), jnp.float32),
            pltpu.VMEM((CHUNK,), jnp.float32),
            pltpu.SemaphoreType.DMA,
        ],
    )
    def sc_kernel(xlin_hbm, src_hbm, dst_hbm, ai_hbm, aj_hbm, nd_hbm,
                  acc_sh, ai_v, aj_v, src_v, dst_v, rows_v, out_v, w_v, sem):
        cid = lax.axis_index("c")
        sid = lax.axis_index("s")
        wid = sid * NCORES + cid

        zeros16 = jnp.zeros((16,), jnp.float32)

        @pl.loop(0, CHUNK)
        def _(r):
            for k in range(DW // 16):
                out_v[r, pl.ds(k * 16, 16)] = zeros16

        base_r = sid * ROWS_PER_TILE
        nfull = ROWS_PER_TILE // CHUNK
        tail = ROWS_PER_TILE - nfull * CHUNK
        for q in range(nfull):
            pltpu.sync_copy(out_v, acc_sh.at[pl.ds(base_r + q * CHUNK, CHUNK)])
        pltpu.sync_copy(out_v.at[pl.ds(0, tail)],
                        acc_sh.at[pl.ds(base_r + nfull * CHUNK, tail)])

        @pl.when(sid == NSUB - 1)
        def _():
            pltpu.sync_copy(out_v.at[pl.ds(0, TAIL_ROWS)],
                            acc_sh.at[pl.ds(NSUB * ROWS_PER_TILE, TAIL_ROWS)])

        pltpu.sync_copy(ai_hbm, ai_v)
        pltpu.sync_copy(aj_hbm, aj_v)

        plsc.subcore_barrier()

        nchunks = jnp.where(wid < NCHUNKS - (NCHUNKS // NTILES) * NTILES,
                            NCHUNKS // NTILES + 1, NCHUNKS // NTILES)

        def chunk_body(k, _):
            chunk = k * NTILES + wid
            base = chunk * CHUNK
            pltpu.sync_copy(src_hbm.at[pl.ds(base, CHUNK)], src_v)
            pltpu.sync_copy(dst_hbm.at[pl.ds(base, CHUNK)], dst_v)
            gcopy = pltpu.async_copy(xlin_hbm.at[src_v], rows_v, sem)
            for b in range(CHUNK // 16):
                sl = pl.ds(b * 16, 16)
                si = src_v[sl]
                di = dst_v[sl]
                s = (plsc.load_gather(ai_v, [di])
                     + plsc.load_gather(aj_v, [si]))
                s = jnp.maximum(s, s * NEG_SLOPE)
                w = jnp.where(si == di, 0.0, jnp.exp(s))
                w_v[sl] = w
            gcopy.wait()

            @pl.loop(0, CHUNK)
            def _(e):
                wb = plsc.load_gather(w_v, [jnp.full((16,), e, jnp.int32)])
                for k2 in range(D // 16):
                    sle = pl.ds(k2 * 16, 16)
                    out_v[e, sle] = rows_v[e, sle] * wb

            pltpu.sync_copy(out_v, acc_sh.at[dst_v], add=True)
            return ()

        lax.fori_loop(0, nchunks, chunk_body, ())

        plsc.subcore_barrier()

        def flush(off, nrows):
            pltpu.sync_copy(acc_sh.at[pl.ds(off, nrows)],
                            out_v.at[pl.ds(0, nrows)])
            pltpu.sync_copy(out_v.at[pl.ds(0, nrows)],
                            nd_hbm.at[cid, pl.ds(off, nrows)])

        for q in range(nfull):
            flush(base_r + q * CHUNK, CHUNK)
        flush(base_r + nfull * CHUNK, tail)

        @pl.when(sid == NSUB - 1)
        def _():
            flush(NSUB * ROWS_PER_TILE, TAIL_ROWS)

    return sc_kernel(xlin, src, dst, ai, aj)


def _sc_denom(src, dst, ai, aj):
    mesh = plsc.VectorSubcoreMesh(core_axis_name="c", subcore_axis_name="s")
    cp = pltpu.CompilerParams()
    if "needs_layout_passes" in pltpu.CompilerParams.__dataclass_fields__:
        cp = dataclasses.replace(cp, needs_layout_passes=False)

    @functools.partial(
        pl.kernel,
        compiler_params=cp,
        out_type=jax.ShapeDtypeStruct((NCORES, N, DW), jnp.float32),
        mesh=mesh,
        scratch_types=[
            pltpu.VMEM_SHARED((N, DW), jnp.float32),
            pltpu.VMEM((N,), jnp.float32),
            pltpu.VMEM((N,), jnp.float32),
            pltpu.VMEM((CHUNK,), jnp.int32),
            pltpu.VMEM((CHUNK,), jnp.int32),
            pltpu.VMEM((CHUNK, DW), jnp.float32),
            pltpu.VMEM((CHUNK,), jnp.float32),
        ],
    )
    def den_kernel(src_hbm, dst_hbm, ai_hbm, aj_hbm, den_hbm,
                   den_sh, ai_v, aj_v, src_v, dst_v, out_v, w_v):
        cid = lax.axis_index("c")
        sid = lax.axis_index("s")
        wid = sid * NCORES + cid

        zeros16 = jnp.zeros((16,), jnp.float32)

        @pl.loop(0, CHUNK)
        def _(r):
            for k in range(DW // 16):
                out_v[r, pl.ds(k * 16, 16)] = zeros16

        base_r = sid * ROWS_PER_TILE
        nfull = ROWS_PER_TILE // CHUNK
        tail = ROWS_PER_TILE - nfull * CHUNK
        for q in range(nfull):
            pltpu.sync_copy(out_v, den_sh.at[pl.ds(base_r + q * CHUNK, CHUNK)])
        pltpu.sync_copy(out_v.at[pl.ds(0, tail)],
                        den_sh.at[pl.ds(base_r + nfull * CHUNK, tail)])

        @pl.when(sid == NSUB - 1)
        def _():
            pltpu.sync_copy(out_v.at[pl.ds(0, TAIL_ROWS)],
                            den_sh.at[pl.ds(NSUB * ROWS_PER_TILE, TAIL_ROWS)])

        pltpu.sync_copy(ai_hbm, ai_v)
        pltpu.sync_copy(aj_hbm, aj_v)

        plsc.subcore_barrier()

        nchunks = jnp.where(wid < NCHUNKS - (NCHUNKS // NTILES) * NTILES,
                            NCHUNKS // NTILES + 1, NCHUNKS // NTILES)

        def chunk_body(k, _):
            chunk = k * NTILES + wid
            base = chunk * CHUNK
            pltpu.sync_copy(src_hbm.at[pl.ds(base, CHUNK)], src_v)
            pltpu.sync_copy(dst_hbm.at[pl.ds(base, CHUNK)], dst_v)
            for b in range(CHUNK // 16):
                sl = pl.ds(b * 16, 16)
                si = src_v[sl]
                di = dst_v[sl]
                s = (plsc.load_gather(ai_v, [di])
                     + plsc.load_gather(aj_v, [si]))
                s = jnp.maximum(s, s * NEG_SLOPE)
                w = jnp.where(si == di, 0.0, jnp.exp(s))
                w_v[sl] = w

            @pl.loop(0, CHUNK)
            def _(e):
                wb = plsc.load_gather(w_v, [jnp.full((16,), e, jnp.int32)])
                for k2 in range(DW // 16):
                    out_v[e, pl.ds(k2 * 16, 16)] = wb

            pltpu.sync_copy(out_v, den_sh.at[dst_v], add=True)
            return ()

        lax.fori_loop(0, nchunks, chunk_body, ())

        plsc.subcore_barrier()

        def flush(off, nrows):
            pltpu.sync_copy(den_sh.at[pl.ds(off, nrows)],
                            out_v.at[pl.ds(0, nrows)])
            pltpu.sync_copy(out_v.at[pl.ds(0, nrows)],
                            den_hbm.at[cid, pl.ds(off, nrows)])

        for q in range(nfull):
            flush(base_r + q * CHUNK, CHUNK)
        flush(base_r + nfull * CHUNK, tail)

        @pl.when(sid == NSUB - 1)
        def _():
            flush(NSUB * ROWS_PER_TILE, TAIL_ROWS)

    return den_kernel(src, dst, ai, aj)


def _epilogue_body(xlin_ref, ai_ref, aj_ref, nd0_ref, nd1_ref,
                   d0_ref, d1_ref, bias_ref, gamma_ref, beta_ref, out_ref):
    s = ai_ref[...] + aj_ref[...]
    s = jnp.maximum(s, s * NEG_SLOPE)
    wself = jnp.exp(s)
    nd0 = nd0_ref[...]
    nd1 = nd1_ref[...]
    numer = nd0[:, :D] + nd1[:, :D] + wself * xlin_ref[...]
    den = d0_ref[...][:, :1] + d1_ref[...][:, :1] + wself
    out = numer / den
    inv_bn = 1.0 / jnp.sqrt(1.0 + 1e-5)
    out = (out + bias_ref[...]) * inv_bn * gamma_ref[...] + beta_ref[...]
    out_ref[...] = jnp.maximum(out, 0.0)


def _epilogue(xlin, ai, aj, nd0, nd1, d0, d1, bias, gamma, beta):
    grid = (N // ROW_BLOCK,)
    row_spec = pl.BlockSpec((ROW_BLOCK, D), lambda j: (j, 0))
    nd_spec = pl.BlockSpec((ROW_BLOCK, DW), lambda j: (j, 0))
    d_spec = pl.BlockSpec((ROW_BLOCK, DW), lambda j: (j, 0))
    col_spec = pl.BlockSpec((ROW_BLOCK, 1), lambda j: (j, 0))
    vec_spec = pl.BlockSpec((1, D), lambda j: (0, 0))
    return pl.pallas_call(
        _epilogue_body,
        grid=grid,
        in_specs=[row_spec, col_spec, col_spec, nd_spec, nd_spec,
                  d_spec, d_spec, vec_spec, vec_spec, vec_spec],
        out_specs=row_spec,
        out_shape=jax.ShapeDtypeStruct((N, D), jnp.float32),
    )(xlin, ai, aj, nd0, nd1, d0, d1, bias, gamma, beta)


def kernel(x, edge_index, embedding, W, att_i, att_j, att_em_i, att_em_j,
           bias, gamma, beta):
    wt = W.T
    vix = att_i.reshape(1, D)
    vie = att_em_i.reshape(1, D)
    vjx = att_j.reshape(1, D)
    vje = att_em_j.reshape(1, D)

    xlin, ai, aj = _prologue(x, embedding, wt, vix, vie, vjx, vje)

    src = edge_index[0]
    dst = edge_index[1]
    nd = _sc_edge(xlin, src, dst, ai.reshape(N), aj.reshape(N))
    den = _sc_denom(src, dst, ai.reshape(N), aj.reshape(N))

    out = _epilogue(xlin, ai, aj, nd[0], nd[1], den[0], den[1],
                    bias.reshape(1, D), gamma.reshape(1, D),
                    beta.reshape(1, D))
    return out


# speedup vs baseline: 10.1845x; 1.1510x over previous
import dataclasses
import functools

import jax
import jax.numpy as jnp
from jax import lax
from jax.experimental import pallas as pl
from jax.experimental.pallas import tpu as pltpu
from jax.experimental.pallas import tpu_sc as plsc

N = 10000
E = 320000
D = 128
DW = 128
NCORES = 2
NSUB = 16
NTILES = NCORES * NSUB
CHUNK = 64
NCHUNKS = E // CHUNK
ROWS_PER_TILE = 624
TAIL_ROWS = N - NSUB * ROWS_PER_TILE
NEG_SLOPE = 0.2
ROW_BLOCK = 1000


def _prologue_body(x_ref, emb_ref, wt_ref, vix_ref, vie_ref, vjx_ref, vje_ref,
                   xlin_ref, ai_ref, aj_ref):
    xl = jnp.dot(x_ref[...], wt_ref[...], preferred_element_type=jnp.float32)
    xlin_ref[...] = xl
    emb = emb_ref[...]
    ai_ref[...] = (jnp.sum(xl * vix_ref[...], axis=1, keepdims=True)
                   + jnp.sum(emb * vie_ref[...], axis=1, keepdims=True))
    aj_ref[...] = (jnp.sum(xl * vjx_ref[...], axis=1, keepdims=True)
                   + jnp.sum(emb * vje_ref[...], axis=1, keepdims=True))


def _prologue(x, emb, wt, vix, vie, vjx, vje):
    grid = (N // ROW_BLOCK,)
    return pl.pallas_call(
        _prologue_body,
        grid=grid,
        in_specs=[
            pl.BlockSpec((ROW_BLOCK, D), lambda j: (j, 0)),
            pl.BlockSpec((ROW_BLOCK, D), lambda j: (j, 0)),
            pl.BlockSpec((D, D), lambda j: (0, 0)),
            pl.BlockSpec((1, D), lambda j: (0, 0)),
            pl.BlockSpec((1, D), lambda j: (0, 0)),
            pl.BlockSpec((1, D), lambda j: (0, 0)),
            pl.BlockSpec((1, D), lambda j: (0, 0)),
        ],
        out_specs=[
            pl.BlockSpec((ROW_BLOCK, D), lambda j: (j, 0)),
            pl.BlockSpec((ROW_BLOCK, 1), lambda j: (j, 0)),
            pl.BlockSpec((ROW_BLOCK, 1), lambda j: (j, 0)),
        ],
        out_shape=[
            jax.ShapeDtypeStruct((N, D), jnp.float32),
            jax.ShapeDtypeStruct((N, 1), jnp.float32),
            jax.ShapeDtypeStruct((N, 1), jnp.float32),
        ],
    )(x, emb, wt, vix, vie, vjx, vje)


def _sc_edge(xlin, src, dst, ai, aj):
    mesh = plsc.VectorSubcoreMesh(core_axis_name="c", subcore_axis_name="s")
    cp = pltpu.CompilerParams()
    if "needs_layout_passes" in pltpu.CompilerParams.__dataclass_fields__:
        cp = dataclasses.replace(cp, needs_layout_passes=False)

    @functools.partial(
        pl.kernel,
        compiler_params=cp,
        out_type=jax.ShapeDtypeStruct((NCORES, N, DW), jnp.float32),
        mesh=mesh,
        scratch_types=[
            pltpu.VMEM_SHARED((N, DW), jnp.float32),
            pltpu.VMEM((N,), jnp.float32),
            pltpu.VMEM((N,), jnp.float32),
            pltpu.VMEM((CHUNK,), jnp.int32),
            pltpu.VMEM((CHUNK,), jnp.int32),
            pltpu.VMEM((CHUNK, D), jnp.float32),
            pltpu.VMEM((CHUNK, DW), jnp.float32),
            pltpu.VMEM((CHUNK,), jnp.float32),
            pltpu.SemaphoreType.DMA,
        ],
    )
    def sc_kernel(xlin_hbm, src_hbm, dst_hbm, ai_hbm, aj_hbm, nd_hbm,
                  acc_sh, ai_v, aj_v, src_v, dst_v, rows_v, out_v, w_v, sem):
        cid = lax.axis_index("c")
        sid = lax.axis_index("s")
        wid = sid * NCORES + cid

        zeros16 = jnp.zeros((16,), jnp.float32)

        @pl.loop(0, CHUNK)
        def _(r):
            for k in range(DW // 16):
                out_v[r, pl.ds(k * 16, 16)] = zeros16

        base_r = sid * ROWS_PER_TILE
        nfull = ROWS_PER_TILE // CHUNK
        tail = ROWS_PER_TILE - nfull * CHUNK
        for q in range(nfull):
            pltpu.sync_copy(out_v, acc_sh.at[pl.ds(base_r + q * CHUNK, CHUNK)])
        pltpu.sync_copy(out_v.at[pl.ds(0, tail)],
                        acc_sh.at[pl.ds(base_r + nfull * CHUNK, tail)])

        @pl.when(sid == NSUB - 1)
        def _():
            pltpu.sync_copy(out_v.at[pl.ds(0, TAIL_ROWS)],
                            acc_sh.at[pl.ds(NSUB * ROWS_PER_TILE, TAIL_ROWS)])

        pltpu.sync_copy(ai_hbm, ai_v)
        pltpu.sync_copy(aj_hbm, aj_v)

        plsc.subcore_barrier()

        nchunks = jnp.where(wid < NCHUNKS - (NCHUNKS // NTILES) * NTILES,
                            NCHUNKS // NTILES + 1, NCHUNKS // NTILES)

        def chunk_body(k, _):
            chunk = k * NTILES + wid
            base = chunk * CHUNK
            pltpu.sync_copy(src_hbm.at[pl.ds(base, CHUNK)], src_v)
            pltpu.sync_copy(dst_hbm.at[pl.ds(base, CHUNK)], dst_v)
            gcopy = pltpu.async_copy(xlin_hbm.at[src_v], rows_v, sem)
            for b in range(CHUNK // 16):
                sl = pl.ds(b * 16, 16)
                si = src_v[sl]
                di = dst_v[sl]
                s = (plsc.load_gather(ai_v, [di])
                     + plsc.load_gather(aj_v, [si]))
                s = jnp.maximum(s, s * NEG_SLOPE)
                w = jnp.where(si == di, 0.0, jnp.exp(s))
                w_v[sl] = w
            gcopy.wait()

            @pl.loop(0, CHUNK)
            def _(e):
                wb = plsc.load_gather(w_v, [jnp.full((16,), e, jnp.int32)])
                for k2 in range(D // 16):
                    sle = pl.ds(k2 * 16, 16)
                    out_v[e, sle] = rows_v[e, sle] * wb

            pltpu.sync_copy(out_v, acc_sh.at[dst_v], add=True)
            return ()

        lax.fori_loop(0, nchunks, chunk_body, ())

        plsc.subcore_barrier()

        def flush(off, nrows):
            pltpu.sync_copy(acc_sh.at[pl.ds(off, nrows)],
                            out_v.at[pl.ds(0, nrows)])
            pltpu.sync_copy(out_v.at[pl.ds(0, nrows)],
                            nd_hbm.at[cid, pl.ds(off, nrows)])

        for q in range(nfull):
            flush(base_r + q * CHUNK, CHUNK)
        flush(base_r + nfull * CHUNK, tail)

        @pl.when(sid == NSUB - 1)
        def _():
            flush(NSUB * ROWS_PER_TILE, TAIL_ROWS)

    return sc_kernel(xlin, src, dst, ai, aj)


def _sc_denom(src, dst, ai, aj):
    mesh = plsc.VectorSubcoreMesh(core_axis_name="c", subcore_axis_name="s")
    cp = pltpu.CompilerParams()
    if "needs_layout_passes" in pltpu.CompilerParams.__dataclass_fields__:
        cp = dataclasses.replace(cp, needs_layout_passes=False)

    DR = 80

    @functools.partial(
        pl.kernel,
        compiler_params=cp,
        out_type=jax.ShapeDtypeStruct((NCORES, DR, D), jnp.float32),
        mesh=mesh,
        scratch_types=[
            pltpu.VMEM_SHARED((DR, D), jnp.float32),
            pltpu.VMEM((N,), jnp.float32),
            pltpu.VMEM((N,), jnp.float32),
            pltpu.VMEM((CHUNK,), jnp.int32),
            pltpu.VMEM((CHUNK,), jnp.int32),
            pltpu.VMEM((DR, D), jnp.float32),
            pltpu.VMEM((DR,), jnp.int32),
        ],
    )
    def den_kernel(src_hbm, dst_hbm, ai_hbm, aj_hbm, den_hbm,
                   den_sh, ai_v, aj_v, src_v, dst_v, dloc_v, ramp_v):
        cid = lax.axis_index("c")
        sid = lax.axis_index("s")
        wid = sid * NCORES + cid

        zeros16 = jnp.zeros((16,), jnp.float32)

        @pl.loop(0, DR)
        def _(r):
            for k in range(D // 16):
                dloc_v[r, pl.ds(k * 16, 16)] = zeros16

        for q in range(DR // 16):
            ramp_v[pl.ds(q * 16, 16)] = lax.iota(jnp.int32, 16) + q * 16

        @pl.when(sid == 0)
        def _():
            pltpu.sync_copy(dloc_v, den_sh)

        pltpu.sync_copy(ai_hbm, ai_v)
        pltpu.sync_copy(aj_hbm, aj_v)

        plsc.subcore_barrier()

        nchunks = jnp.where(wid < NCHUNKS - (NCHUNKS // NTILES) * NTILES,
                            NCHUNKS // NTILES + 1, NCHUNKS // NTILES)

        def chunk_body(k, _):
            chunk = k * NTILES + wid
            base = chunk * CHUNK
            pltpu.sync_copy(src_hbm.at[pl.ds(base, CHUNK)], src_v)
            pltpu.sync_copy(dst_hbm.at[pl.ds(base, CHUNK)], dst_v)
            for b in range(CHUNK // 16):
                sl = pl.ds(b * 16, 16)
                si = src_v[sl]
                di = dst_v[sl]
                s = (plsc.load_gather(ai_v, [di])
                     + plsc.load_gather(aj_v, [si]))
                s = jnp.maximum(s, s * NEG_SLOPE)
                w = jnp.where(si == di, 0.0, jnp.exp(s))
                plsc.addupdate_scatter(
                    dloc_v, [lax.shift_right_logical(di, 7), di & 127], w)
            return ()

        lax.fori_loop(0, nchunks, chunk_body, ())

        plsc.subcore_barrier()
        pltpu.sync_copy(dloc_v, den_sh.at[ramp_v], add=True)
        plsc.subcore_barrier()

        @pl.when(sid == 0)
        def _():
            pltpu.sync_copy(den_sh, dloc_v)
            pltpu.sync_copy(dloc_v, den_hbm.at[cid])

    return den_kernel(src, dst, ai, aj)


def _epilogue_body(xlin_ref, ai_ref, aj_ref, nd0_ref, nd1_ref,
                   d0_ref, d1_ref, bias_ref, gamma_ref, beta_ref, out_ref):
    s = ai_ref[...] + aj_ref[...]
    s = jnp.maximum(s, s * NEG_SLOPE)
    wself = jnp.exp(s)
    nd0 = nd0_ref[...]
    nd1 = nd1_ref[...]
    numer = nd0[:, :D] + nd1[:, :D] + wself * xlin_ref[...]
    den = d0_ref[...] + d1_ref[...] + wself
    out = numer / den
    inv_bn = 1.0 / jnp.sqrt(1.0 + 1e-5)
    out = (out + bias_ref[...]) * inv_bn * gamma_ref[...] + beta_ref[...]
    out_ref[...] = jnp.maximum(out, 0.0)


def _epilogue(xlin, ai, aj, nd0, nd1, d0, d1, bias, gamma, beta):
    grid = (N // ROW_BLOCK,)
    row_spec = pl.BlockSpec((ROW_BLOCK, D), lambda j: (j, 0))
    nd_spec = pl.BlockSpec((ROW_BLOCK, DW), lambda j: (j, 0))
    d_spec = pl.BlockSpec((ROW_BLOCK, 1), lambda j: (j, 0))
    col_spec = pl.BlockSpec((ROW_BLOCK, 1), lambda j: (j, 0))
    vec_spec = pl.BlockSpec((1, D), lambda j: (0, 0))
    return pl.pallas_call(
        _epilogue_body,
        grid=grid,
        in_specs=[row_spec, col_spec, col_spec, nd_spec, nd_spec,
                  d_spec, d_spec, vec_spec, vec_spec, vec_spec],
        out_specs=row_spec,
        out_shape=jax.ShapeDtypeStruct((N, D), jnp.float32),
    )(xlin, ai, aj, nd0, nd1, d0, d1, bias, gamma, beta)


def kernel(x, edge_index, embedding, W, att_i, att_j, att_em_i, att_em_j,
           bias, gamma, beta):
    wt = W.T
    vix = att_i.reshape(1, D)
    vie = att_em_i.reshape(1, D)
    vjx = att_j.reshape(1, D)
    vje = att_em_j.reshape(1, D)

    xlin, ai, aj = _prologue(x, embedding, wt, vix, vie, vjx, vje)

    src = edge_index[0]
    dst = edge_index[1]
    nd = _sc_edge(xlin, src, dst, ai.reshape(N), aj.reshape(N))
    den = _sc_denom(src, dst, ai.reshape(N), aj.reshape(N))
    denf = den.reshape(NCORES, 80 * D)[:, :N]

    out = _epilogue(xlin, ai, aj, nd[0], nd[1],
                    denf[0].reshape(N, 1), denf[1].reshape(N, 1),
                    bias.reshape(1, D), gamma.reshape(1, D),
                    beta.reshape(1, D))
    return out
